# baseline (speedup 1.0000x reference)
import jax
import jax.numpy as jnp
from jax.experimental import pallas as pl
from jax.experimental.pallas import tpu as pltpu

_NBINS = 38
_IBLK = 32
_NBUF = 4
_JCHUNK = 128
_FLATC = _JCHUNK * _NBINS


def _onehot_kernel(at_ref, bt_ref, aux_ref, w_ref, kflat_ref, out_hbm,
                   scratch, sems):
    n = bt_ref.shape[2]
    nblk = n // _IBLK
    step = pl.program_id(0)
    nsteps = pl.num_programs(0)
    b = step // nblk
    i = step % nblk
    slot = jax.lax.rem(step, _NBUF)

    def _copy(s, dst_b, dst_i):
        return pltpu.make_async_copy(
            scratch.at[s],
            out_hbm.at[dst_b, pl.ds(dst_i * _IBLK, _IBLK), :],
            sems.at[s],
        )

    @pl.when(step >= _NBUF)
    def _wait_slot():
        _copy(slot, b, i).wait()

    d2 = None
    for c in range(3):
        a = at_ref[0, 0, :, c : c + 1]
        bb = bt_ref[0, c : c + 1, :]
        diff = a - bb
        d2 = diff * diff if d2 is None else d2 + diff * diff
    d = jnp.sqrt(d2)
    start = aux_ref[1:2, 0:1]
    inv_step = aux_ref[1:2, 1:2]
    bidx = jnp.clip(jnp.floor((d - start) * inv_step), 0.0, _NBINS - 1.0)
    bidx16 = bidx.astype(jnp.bfloat16)
    kflat = kflat_ref[0:1, :]
    for c in range(n // _JCHUNK):
        bc = bidx16[:, c * _JCHUNK : (c + 1) * _JCHUNK]
        m = jnp.dot(bc, w_ref[:, :], preferred_element_type=jnp.float32)
        scratch[slot, :, c * _FLATC : (c + 1) * _FLATC] = jnp.where(
            m == kflat, 1.0, 0.0
        )

    _copy(slot, b, i).start()

    @pl.when(step == nsteps - 1)
    def _drain():
        for s in range(_NBUF):
            _copy(s, b, i).wait()


def kernel(CB_coords, v_bins):
    nbatch, n, _ = CB_coords.shape
    nblk = n // _IBLK
    coords_p = jnp.pad(CB_coords, ((0, 0), (0, 0), (0, 5)))
    coords_rows = coords_p.reshape(nbatch, nblk, _IBLK, 8)
    coords_t = jnp.pad(
        jnp.transpose(CB_coords, (0, 2, 1)), ((0, 0), (0, 5), (0, 0))
    )
    aux = jnp.zeros((8, 128), jnp.float32)
    aux = aux.at[1, 0].set(v_bins[0])
    aux = aux.at[1, 1].set(1.0 / (v_bins[1] - v_bins[0]))
    p = jnp.arange(_FLATC, dtype=jnp.int32)
    w = (p[None, :] // _NBINS == jnp.arange(_JCHUNK, dtype=jnp.int32)[:, None])
    w = w.astype(jnp.bfloat16)
    kflat = (p % _NBINS).astype(jnp.float32)[None, :]

    grid = (nbatch * nblk,)
    out = pl.pallas_call(
        _onehot_kernel,
        grid=grid,
        in_specs=[
            pl.BlockSpec((1, 1, _IBLK, 8), lambda s: (s // (n // _IBLK), s % (n // _IBLK), 0, 0)),
            pl.BlockSpec((1, 8, n), lambda s: (s // (n // _IBLK), 0, 0)),
            pl.BlockSpec((8, 128), lambda s: (0, 0)),
            pl.BlockSpec((_JCHUNK, _FLATC), lambda s: (0, 0)),
            pl.BlockSpec((1, _FLATC), lambda s: (0, 0)),
        ],
        out_specs=pl.BlockSpec(memory_space=pl.ANY),
        out_shape=jax.ShapeDtypeStruct((nbatch, n, n * _NBINS), jnp.float32),
        scratch_shapes=[
            pltpu.VMEM((_NBUF, _IBLK, n * _NBINS), jnp.float32),
            pltpu.SemaphoreType.DMA((_NBUF,)),
        ],
    )(coords_rows, coords_t, aux, w, kflat)
    return out.reshape(nbatch, n, n, _NBINS)
